# baseline (speedup 1.0000x reference)
import functools

import jax
import jax.numpy as jnp
from jax import lax
from jax.experimental import pallas as pl
from jax.experimental.pallas import tpu as pltpu
from jax.experimental.pallas import tpu_sc as plsc

VOCAB = 1000000
DIM = 64
BATCH = 4096
SEQ = 200
B = BATCH * SEQ
NC = 2
NS = 16
NW = NC * NS
BPW = B // NW
CH = 128
NCHUNK = BPW // CH
NSLOT = 5
AHEAD = 4


def _emb_body(x_hbm, table_hbm, out_hbm, idx_v, rows_v, *sems):
    gsems = sems[:NSLOT]
    osems = sems[NSLOT:]
    wid = lax.axis_index("s") * NC + lax.axis_index("c")
    base = wid * BPW

    pltpu.sync_copy(x_hbm.at[pl.ds(base, BPW)], idx_v)

    def fire_gather(k, slot):
        pltpu.async_copy(
            table_hbm.at[idx_v.at[pl.ds(k * CH, CH)]], rows_v.at[slot],
            gsems[slot])

    def wait_gather(k, slot):
        pltpu.make_async_copy(
            table_hbm.at[idx_v.at[pl.ds(k * CH, CH)]], rows_v.at[slot],
            gsems[slot]).wait()

    def fire_out(k, slot):
        pltpu.async_copy(
            rows_v.at[slot], out_hbm.at[pl.ds(base + k * CH, CH)],
            osems[slot])

    def wait_out(k, slot):
        pltpu.make_async_copy(
            rows_v.at[slot], out_hbm.at[pl.ds(base + k * CH, CH)],
            osems[slot]).wait()

    for u in range(AHEAD):
        fire_gather(u, u)

    def step(i, carry):
        for u in range(NSLOT):
            k = NSLOT * i + u
            wait_gather(k, u)
            fire_out(k, u)
            ka = k + AHEAD
            sa = (u + AHEAD) % NSLOT

            @pl.when(ka < NCHUNK)
            def _():
                @pl.when(ka >= NSLOT)
                def _():
                    wait_out(ka - NSLOT, sa)

                fire_gather(ka, sa)
        return carry

    lax.fori_loop(0, NCHUNK // NSLOT, step, 0)
    for u in range(NSLOT):
        wait_out(NCHUNK - NSLOT + u, (NCHUNK - NSLOT + u) % NSLOT)


@jax.jit
def _emb(xf, tablep):
    mesh = plsc.VectorSubcoreMesh(core_axis_name="c", subcore_axis_name="s")
    kern = functools.partial(
        pl.kernel,
        out_type=jax.ShapeDtypeStruct((B, 2 * DIM), jnp.float32),
        mesh=mesh,
        scratch_types=[
            pltpu.VMEM((BPW,), jnp.int32),
            pltpu.VMEM((NSLOT, CH, 128), jnp.float32),
        ] + [pltpu.SemaphoreType.DMA] * (2 * NSLOT),
        compiler_params=pltpu.CompilerParams(
            use_tc_tiling_on_sc=True, needs_layout_passes=False),
    )(_emb_body)
    return kern(xf, tablep)


def kernel(x, table):
    xf = x.astype(jnp.int32).reshape(B)
    tablep = jnp.pad(table, ((0, 0), (0, DIM)))
    out = _emb(xf, tablep)
    return out[:, :DIM].reshape(BATCH, SEQ, DIM)


# speedup vs baseline: 1.0021x; 1.0021x over previous
import functools

import jax
import jax.numpy as jnp
from jax import lax
from jax.experimental import pallas as pl
from jax.experimental.pallas import tpu as pltpu
from jax.experimental.pallas import tpu_sc as plsc

VOCAB = 1000000
DIM = 64
BATCH = 4096
SEQ = 200
B = BATCH * SEQ
NC = 2
NS = 16
NW = NC * NS
BPW = B // NW
CH = 128
NCHUNK = BPW // CH
NSLOT = 4
AHEAD = 3


def _emb_body(x_hbm, table_hbm, out_hbm, idx_v, rows_v, *sems):
    gsems = sems[:NSLOT]
    osems = sems[NSLOT:]
    wid = lax.axis_index("s") * NC + lax.axis_index("c")
    base = wid * BPW

    pltpu.sync_copy(x_hbm.at[pl.ds(base, BPW)], idx_v)

    def fire_gather(k, slot):
        pltpu.async_copy(
            table_hbm.at[idx_v.at[pl.ds(k * CH, CH)]], rows_v.at[slot],
            gsems[slot])

    def wait_gather(k, slot):
        pltpu.make_async_copy(
            table_hbm.at[idx_v.at[pl.ds(k * CH, CH)]], rows_v.at[slot],
            gsems[slot]).wait()

    def fire_out(k, slot):
        pltpu.async_copy(
            rows_v.at[slot], out_hbm.at[pl.ds(base + k * CH, CH)],
            osems[slot])

    def wait_out(k, slot):
        pltpu.make_async_copy(
            rows_v.at[slot], out_hbm.at[pl.ds(base + k * CH, CH)],
            osems[slot]).wait()

    for u in range(AHEAD):
        fire_gather(u, u)

    def step(i, carry):
        for u in range(NSLOT):
            k = NSLOT * i + u
            wait_gather(k, u)
            fire_out(k, u)
            ka = k + AHEAD
            sa = (u + AHEAD) % NSLOT

            @pl.when(ka < NCHUNK)
            def _():
                @pl.when(ka >= NSLOT)
                def _():
                    wait_out(ka - NSLOT, sa)

                fire_gather(ka, sa)
        return carry

    lax.fori_loop(0, NCHUNK // NSLOT, step, 0)
    for u in range(NSLOT):
        wait_out(NCHUNK - NSLOT + u, (NCHUNK - NSLOT + u) % NSLOT)


@jax.jit
def _emb(xf, tablep):
    mesh = plsc.VectorSubcoreMesh(core_axis_name="c", subcore_axis_name="s")
    kern = functools.partial(
        pl.kernel,
        out_type=jax.ShapeDtypeStruct((B, 2 * DIM), jnp.float32),
        mesh=mesh,
        scratch_types=[
            pltpu.VMEM((BPW,), jnp.int32),
            pltpu.VMEM((NSLOT, CH, 128), jnp.float32),
        ] + [pltpu.SemaphoreType.DMA] * (2 * NSLOT),
        compiler_params=pltpu.CompilerParams(
            use_tc_tiling_on_sc=True, needs_layout_passes=False),
    )(_emb_body)
    return kern(xf, tablep)


def kernel(x, table):
    xf = x.astype(jnp.int32).reshape(B)
    tablep = jnp.pad(table, ((0, 0), (0, DIM)))
    out = _emb(xf, tablep)
    return out[:, :DIM].reshape(BATCH, SEQ, DIM)
